# baseline (speedup 1.0000x reference)
import functools

import jax
import jax.numpy as jnp
from jax import lax
from jax.experimental import pallas as pl
from jax.experimental.pallas import tpu as pltpu
from jax.experimental.pallas import tpu_sc as plsc

N = 10000
NPAD = 10240
CIN = 6
C = 128
K = 16
DEPTH = 6

NC = 2
NS = 16
NW = NC * NS
RPT = NPAD // NW
GR = 8
NG = RPT // GR

BIG_I = 2**30
INF = 3e38


KNN_R = 256
KNN_NB = NPAD // KNN_R


NCLS = 128
NPOS = NPAD // NCLS
TOPT = 4


def _knn_body(xt_ref, xct_ref, idx_ref, d_ref):
    R = KNN_R
    xr = xt_ref[...]
    xct = xct_ref[...]
    sqc = jnp.sum(xct * xct, axis=0, keepdims=True)
    sqr = jnp.sum(xr * xr, axis=1, keepdims=True)
    prod = jnp.dot(xr, xct, preferred_element_type=jnp.float32)
    dist = sqr + sqc - 2.0 * prod

    pos3 = lax.broadcasted_iota(jnp.int32, (R, NPOS, NCLS), 1)
    lane3 = lax.broadcasted_iota(jnp.int32, (R, NPOS, NCLS), 2)
    col3 = pos3 * NCLS + lane3
    d_ref[...] = jnp.where(col3 >= N, jnp.float32(INF),
                           dist.reshape(R, NPOS, NCLS))

    vals = []
    cols = []
    lane2 = lax.broadcasted_iota(jnp.int32, (R, NCLS), 1)
    for _ in range(TOPT):
        d = d_ref[...]
        m = jnp.min(d, axis=1)
        pc = jnp.where(d <= m[:, None, :], pos3, BIG_I)
        p = jnp.min(pc, axis=1)
        d_ref[...] = jnp.where((pos3 == p[:, None, :])
                               & (d <= m[:, None, :]), INF, d)
        vals.append(m)
        cols.append(p * NCLS + lane2)

    lane_k = lax.broadcasted_iota(jnp.int32, (R, K), 1)
    vcur, icur = vals[0], cols[0]
    ptr = jnp.zeros((R, NCLS), jnp.int32)
    acc = jnp.zeros((R, K), jnp.int32)
    for k in range(K):
        m = jnp.min(vcur, axis=1, keepdims=True)
        cc = jnp.where(vcur <= m, lane2, BIG_I)
        jc = jnp.min(cc, axis=1, keepdims=True)
        sel = lane2 == jc
        colk = jnp.min(jnp.where(sel, icur, BIG_I), axis=1, keepdims=True)
        acc = jnp.where(lane_k == k, colk, acc)
        ptrn = ptr + sel.astype(jnp.int32)
        vnext = jnp.full((R, NCLS), INF, jnp.float32)
        inext = jnp.zeros((R, NCLS), jnp.int32)
        for t in range(1, TOPT):
            vnext = jnp.where(ptrn == t, vals[t], vnext)
            inext = jnp.where(ptrn == t, cols[t], inext)
        vcur = jnp.where(sel, vnext, vcur)
        icur = jnp.where(sel, inext, icur)
        ptr = ptrn
    idx_ref[...] = acc


_knn_call = pl.pallas_call(
    _knn_body,
    grid=(KNN_NB,),
    in_specs=[
        pl.BlockSpec((KNN_R, 8), lambda i: (i, 0)),
        pl.BlockSpec((8, NPAD), lambda i: (0, 0)),
    ],
    out_specs=pl.BlockSpec((KNN_R, K), lambda i: (i, 0)),
    out_shape=jax.ShapeDtypeStruct((NPAD, K), jnp.int32),
    scratch_shapes=[pltpu.VMEM((KNN_R, NPOS, NCLS), jnp.float32)],
)


_gather_cache = {}


def _get_gather_max(key):
    if key not in _gather_cache:
        if isinstance(key, tuple):
            _gather_cache[key] = _make_gather_max(*key)
        else:
            _gather_cache[key] = _make_gather_max(key)
    return _gather_cache[key]


def _make_gather_max(cdim, kk=K):
    gr = 128 // kk
    ng = RPT // gr
    mesh = plsc.VectorSubcoreMesh(
        core_axis_name="c", subcore_axis_name="s",
        num_cores=NC, num_subcores=NS)

    @functools.partial(
        pl.kernel,
        out_type=jax.ShapeDtypeStruct((NPAD, cdim), jnp.float32),
        mesh=mesh,
        scratch_types=[
            pltpu.VMEM((RPT * kk,), jnp.int32),
            pltpu.VMEM((gr * kk, cdim), jnp.float32),
            pltpu.VMEM((gr * kk, cdim), jnp.float32),
            pltpu.VMEM((RPT, cdim), jnp.float32),
            pltpu.SemaphoreType.DMA,
            pltpu.SemaphoreType.DMA,
        ],
    )
    def gather_max(feat_hbm, idxf_hbm, out_hbm, idxv, buf0, buf1, aggv,
                   sem0, sem1):
        wid = lax.axis_index("s") * NC + lax.axis_index("c")
        base = wid * RPT
        pltpu.sync_copy(idxf_hbm.at[pl.ds(base * kk, RPT * kk)], idxv)

        bufs = (buf0, buf1)
        sems = (sem0, sem1)

        def start(g, b):
            pltpu.async_copy(
                feat_hbm.at[idxv.at[pl.ds(g * (gr * kk), gr * kk)]],
                bufs[b], sems[b])

        def compute(g, b):
            buf = bufs[b]
            pltpu.make_async_copy(feat_hbm.at[pl.ds(0, gr * kk)], buf,
                                  sems[b]).wait()

            def row(r, _):
                def chunk(c, _2):
                    acc = buf[r * kk, pl.ds(c * 16, 16)]
                    for k in range(1, kk):
                        acc = jnp.maximum(acc, buf[r * kk + k, pl.ds(c * 16, 16)])
                    aggv[g * gr + r, pl.ds(c * 16, 16)] = acc
                    return 0

                lax.fori_loop(0, cdim // 16, chunk, 0)
                return 0

            lax.fori_loop(0, gr, row, 0)

        start(0, 0)

        def pair(g2, _):
            g = g2 * 2
            start(g + 1, 1)
            compute(g, 0)

            @pl.when(g2 + 1 < ng // 2)
            def _():
                start(g + 2, 0)

            compute(g + 1, 1)
            return 0

        lax.fori_loop(0, ng // 2, pair, 0)
        pltpu.sync_copy(aggv, out_hbm.at[pl.ds(base, RPT)])

    return gather_max


def _conv0_body(xt_ref, agg_ref, wa_ref, wb_ref, b_ref, out_ref):
    h = jnp.dot(xt_ref[...], wa_ref[...], preferred_element_type=jnp.float32)
    h += jnp.dot(agg_ref[...].astype(jnp.float32), wb_ref[...],
                 preferred_element_type=jnp.float32)
    out_ref[...] = h + b_ref[...]


_conv0_call = pl.pallas_call(
    _conv0_body,
    out_shape=jax.ShapeDtypeStruct((NPAD, C), jnp.float32),
)


def _norm_relu(h):
    rowmask = (lax.broadcasted_iota(jnp.int32, (NPAD, 1), 0) < N)
    hm = jnp.where(rowmask, h, 0.0)
    inv_n = jnp.float32(1.0 / N)
    m = jnp.sum(hm, axis=0, keepdims=True) * inv_n
    ex2 = jnp.sum(hm * hm, axis=0, keepdims=True) * inv_n
    v0 = ex2 - m * m
    s = lax.rsqrt(v0 + 1e-3)
    r = v0 * s * s
    s2 = lax.rsqrt(r + 1e-5)
    return (h - m) * (s * s2)


def _conv_a_body(f_ref, agg_ref, w1_ref, w2_ref, b_ref, g_ref, bt_ref, out_ref):
    h = jnp.dot(f_ref[...], w1_ref[...], preferred_element_type=jnp.float32)
    h += jnp.dot(agg_ref[...].astype(jnp.float32), w2_ref[...],
                 preferred_element_type=jnp.float32)
    h += b_ref[...]
    y = _norm_relu(h) * g_ref[...] + bt_ref[...]
    out_ref[...] = jnp.maximum(y, 0.0)


def _conv_b_body(f_ref, agg_ref, w1_ref, w2_ref, b_ref, g_ref, bt_ref,
                 res_ref, out_ref):
    h = jnp.dot(f_ref[...], w1_ref[...], preferred_element_type=jnp.float32)
    h += jnp.dot(agg_ref[...].astype(jnp.float32), w2_ref[...],
                 preferred_element_type=jnp.float32)
    h += b_ref[...]
    y = _norm_relu(h) * g_ref[...] + bt_ref[...]
    out_ref[...] = res_ref[...] + jnp.maximum(y, 0.0)


_conv_a_call = pl.pallas_call(
    _conv_a_body, out_shape=jax.ShapeDtypeStruct((NPAD, C), jnp.float32))
_conv_b_call = pl.pallas_call(
    _conv_b_body, out_shape=jax.ShapeDtypeStruct((NPAD, C), jnp.float32))


def kernel(x, W0, b0, Wa, ba, ga, bta, Wb, bb, gb, btb):
    xt = jnp.transpose(x[0])
    xt8 = jnp.pad(xt, ((0, NPAD - N), (0, 8 - CIN)))
    xct = jnp.transpose(xt8)

    idx = _knn_call(xt8, xct)
    idx_flat = idx.reshape(NPAD * K)

    xt16 = jnp.pad(xt, ((0, NPAD - N), (0, 16 - CIN)))
    xtw = jnp.pad(xt, ((0, NPAD - N), (0, C - CIN)))
    agg0 = _get_gather_max(C)(xtw, idx_flat)[:, :16]

    w0a = jnp.pad(jnp.transpose(W0[:, :CIN]), ((0, 16 - CIN), (0, 0)))
    w0b = jnp.pad(jnp.transpose(W0[:, CIN:]), ((0, 16 - CIN), (0, 0)))
    feat = _conv0_call(xt16, agg0, w0a, w0b, b0[None, :])

    for i in range(DEPTH):
        agg = _get_gather_max(C)(feat, idx_flat)
        h = _conv_a_call(feat, agg,
                         jnp.transpose(Wa[i, :, :C]), jnp.transpose(Wa[i, :, C:]),
                         ba[i][None, :], ga[i][None, :], bta[i][None, :])
        aggb = _get_gather_max(C)(h, idx_flat)
        feat = _conv_b_call(h, aggb,
                            jnp.transpose(Wb[i, :, :C]), jnp.transpose(Wb[i, :, C:]),
                            bb[i][None, :], gb[i][None, :], btb[i][None, :],
                            feat)

    return jnp.transpose(feat[:N])[None]


# speedup vs baseline: 1.0024x; 1.0024x over previous
import functools

import jax
import jax.numpy as jnp
from jax import lax
from jax.experimental import pallas as pl
from jax.experimental.pallas import tpu as pltpu
from jax.experimental.pallas import tpu_sc as plsc

N = 10000
NPAD = 10240
CIN = 6
C = 128
K = 16
DEPTH = 6

NC = 2
NS = 16
NW = NC * NS
RPT = NPAD // NW
GR = 8
NG = RPT // GR

BIG_I = 2**30
INF = 3e38


KNN_R = 256
KNN_NB = NPAD // KNN_R


NCLS = 128
NPOS = NPAD // NCLS
TOPT = 4


def _knn_body(xt_ref, xct_ref, idx_ref, d_ref):
    R = KNN_R
    xr = xt_ref[...]
    xct = xct_ref[...]
    sqc = jnp.sum(xct * xct, axis=0, keepdims=True)
    sqr = jnp.sum(xr * xr, axis=1, keepdims=True)
    prod = jnp.dot(xr, xct, preferred_element_type=jnp.float32)
    dist = sqr + sqc - 2.0 * prod

    pos3 = lax.broadcasted_iota(jnp.int32, (R, NPOS, NCLS), 1)
    lane3 = lax.broadcasted_iota(jnp.int32, (R, NPOS, NCLS), 2)
    col3 = pos3 * NCLS + lane3
    d_ref[...] = jnp.where(col3 >= N, jnp.float32(INF),
                           dist.reshape(R, NPOS, NCLS))

    vals = []
    cols = []
    lane2 = lax.broadcasted_iota(jnp.int32, (R, NCLS), 1)
    for _ in range(TOPT):
        d = d_ref[...]
        m = jnp.min(d, axis=1)
        pc = jnp.where(d <= m[:, None, :], pos3, BIG_I)
        p = jnp.min(pc, axis=1)
        d_ref[...] = jnp.where((pos3 == p[:, None, :])
                               & (d <= m[:, None, :]), INF, d)
        vals.append(m)
        cols.append(p * NCLS + lane2)

    lane_k = lax.broadcasted_iota(jnp.int32, (R, K), 1)
    vcur, icur = vals[0], cols[0]
    ptr = jnp.zeros((R, NCLS), jnp.int32)
    acc = jnp.zeros((R, K), jnp.int32)
    for k in range(K):
        m = jnp.min(vcur, axis=1, keepdims=True)
        cc = jnp.where(vcur <= m, lane2, BIG_I)
        jc = jnp.min(cc, axis=1, keepdims=True)
        sel = lane2 == jc
        colk = jnp.min(jnp.where(sel, icur, BIG_I), axis=1, keepdims=True)
        acc = jnp.where(lane_k == k, colk, acc)
        ptrn = ptr + sel.astype(jnp.int32)
        vnext = jnp.full((R, NCLS), INF, jnp.float32)
        inext = jnp.zeros((R, NCLS), jnp.int32)
        for t in range(1, TOPT):
            vnext = jnp.where(ptrn == t, vals[t], vnext)
            inext = jnp.where(ptrn == t, cols[t], inext)
        vcur = jnp.where(sel, vnext, vcur)
        icur = jnp.where(sel, inext, icur)
        ptr = ptrn
    idx_ref[...] = acc


_knn_call = pl.pallas_call(
    _knn_body,
    grid=(KNN_NB,),
    in_specs=[
        pl.BlockSpec((KNN_R, 8), lambda i: (i, 0)),
        pl.BlockSpec((8, NPAD), lambda i: (0, 0)),
    ],
    out_specs=pl.BlockSpec((KNN_R, K), lambda i: (i, 0)),
    out_shape=jax.ShapeDtypeStruct((NPAD, K), jnp.int32),
    scratch_shapes=[pltpu.VMEM((KNN_R, NPOS, NCLS), jnp.float32)],
)


_gather_cache = {}


def _get_gather_max(key):
    if key not in _gather_cache:
        if isinstance(key, tuple):
            _gather_cache[key] = _make_gather_max(*key)
        else:
            _gather_cache[key] = _make_gather_max(key)
    return _gather_cache[key]


def _make_gather_max(cdim, kk=K):
    gr = 128 // kk
    ng = RPT // gr
    mesh = plsc.VectorSubcoreMesh(
        core_axis_name="c", subcore_axis_name="s",
        num_cores=NC, num_subcores=NS)

    @functools.partial(
        pl.kernel,
        out_type=jax.ShapeDtypeStruct((NPAD, cdim), jnp.float32),
        mesh=mesh,
        scratch_types=[
            pltpu.VMEM((RPT * kk,), jnp.int32),
            pltpu.VMEM((gr * kk, cdim), jnp.float32),
            pltpu.VMEM((gr * kk, cdim), jnp.float32),
            pltpu.VMEM((RPT, cdim), jnp.float32),
            pltpu.SemaphoreType.DMA,
            pltpu.SemaphoreType.DMA,
        ],
    )
    def gather_max(feat_hbm, idxf_hbm, out_hbm, idxv, buf0, buf1, aggv,
                   sem0, sem1):
        wid = lax.axis_index("s") * NC + lax.axis_index("c")
        base = wid * RPT
        pltpu.sync_copy(idxf_hbm.at[pl.ds(base * kk, RPT * kk)], idxv)

        bufs = (buf0, buf1)
        sems = (sem0, sem1)

        def start(g, b):
            pltpu.async_copy(
                feat_hbm.at[idxv.at[pl.ds(g * (gr * kk), gr * kk)]],
                bufs[b], sems[b])

        def compute(g, b):
            buf = bufs[b]
            pltpu.make_async_copy(feat_hbm.at[pl.ds(0, gr * kk)], buf,
                                  sems[b]).wait()

            def row(r, _):
                for c in range(cdim // 16):
                    acc = buf[r * kk, pl.ds(c * 16, 16)]
                    for k in range(1, kk):
                        acc = jnp.maximum(acc, buf[r * kk + k, pl.ds(c * 16, 16)])
                    aggv[g * gr + r, pl.ds(c * 16, 16)] = acc
                return 0

            lax.fori_loop(0, gr, row, 0)

        start(0, 0)

        def pair(g2, _):
            g = g2 * 2
            start(g + 1, 1)
            compute(g, 0)

            @pl.when(g2 + 1 < ng // 2)
            def _():
                start(g + 2, 0)

            compute(g + 1, 1)
            return 0

        lax.fori_loop(0, ng // 2, pair, 0)
        pltpu.sync_copy(aggv, out_hbm.at[pl.ds(base, RPT)])

    return gather_max


def _conv0_body(xt_ref, agg_ref, wa_ref, wb_ref, b_ref, out_ref):
    h = jnp.dot(xt_ref[...], wa_ref[...], preferred_element_type=jnp.float32)
    h += jnp.dot(agg_ref[...].astype(jnp.float32), wb_ref[...],
                 preferred_element_type=jnp.float32)
    out_ref[...] = h + b_ref[...]


_conv0_call = pl.pallas_call(
    _conv0_body,
    out_shape=jax.ShapeDtypeStruct((NPAD, C), jnp.float32),
)


def _norm_relu(h):
    rowmask = (lax.broadcasted_iota(jnp.int32, (NPAD, 1), 0) < N)
    hm = jnp.where(rowmask, h, 0.0)
    inv_n = jnp.float32(1.0 / N)
    m = jnp.sum(hm, axis=0, keepdims=True) * inv_n
    ex2 = jnp.sum(hm * hm, axis=0, keepdims=True) * inv_n
    v0 = ex2 - m * m
    s = lax.rsqrt(v0 + 1e-3)
    r = v0 * s * s
    s2 = lax.rsqrt(r + 1e-5)
    return (h - m) * (s * s2)


def _conv_a_body(f_ref, agg_ref, w1_ref, w2_ref, b_ref, g_ref, bt_ref, out_ref):
    h = jnp.dot(f_ref[...], w1_ref[...], preferred_element_type=jnp.float32)
    h += jnp.dot(agg_ref[...].astype(jnp.float32), w2_ref[...],
                 preferred_element_type=jnp.float32)
    h += b_ref[...]
    y = _norm_relu(h) * g_ref[...] + bt_ref[...]
    out_ref[...] = jnp.maximum(y, 0.0)


def _conv_b_body(f_ref, agg_ref, w1_ref, w2_ref, b_ref, g_ref, bt_ref,
                 res_ref, out_ref):
    h = jnp.dot(f_ref[...], w1_ref[...], preferred_element_type=jnp.float32)
    h += jnp.dot(agg_ref[...].astype(jnp.float32), w2_ref[...],
                 preferred_element_type=jnp.float32)
    h += b_ref[...]
    y = _norm_relu(h) * g_ref[...] + bt_ref[...]
    out_ref[...] = res_ref[...] + jnp.maximum(y, 0.0)


_conv_a_call = pl.pallas_call(
    _conv_a_body, out_shape=jax.ShapeDtypeStruct((NPAD, C), jnp.float32))
_conv_b_call = pl.pallas_call(
    _conv_b_body, out_shape=jax.ShapeDtypeStruct((NPAD, C), jnp.float32))


def kernel(x, W0, b0, Wa, ba, ga, bta, Wb, bb, gb, btb):
    xt = jnp.transpose(x[0])
    xt8 = jnp.pad(xt, ((0, NPAD - N), (0, 8 - CIN)))
    xct = jnp.transpose(xt8)

    idx = _knn_call(xt8, xct)
    idx_flat = idx.reshape(NPAD * K)

    xt16 = jnp.pad(xt, ((0, NPAD - N), (0, 16 - CIN)))
    xtw = jnp.pad(xt, ((0, NPAD - N), (0, C - CIN)))
    agg0 = _get_gather_max(C)(xtw, idx_flat)[:, :16]

    w0a = jnp.pad(jnp.transpose(W0[:, :CIN]), ((0, 16 - CIN), (0, 0)))
    w0b = jnp.pad(jnp.transpose(W0[:, CIN:]), ((0, 16 - CIN), (0, 0)))
    feat = _conv0_call(xt16, agg0, w0a, w0b, b0[None, :])

    for i in range(DEPTH):
        agg = _get_gather_max(C)(feat, idx_flat)
        h = _conv_a_call(feat, agg,
                         jnp.transpose(Wa[i, :, :C]), jnp.transpose(Wa[i, :, C:]),
                         ba[i][None, :], ga[i][None, :], bta[i][None, :])
        aggb = _get_gather_max(C)(h, idx_flat)
        feat = _conv_b_call(h, aggb,
                            jnp.transpose(Wb[i, :, :C]), jnp.transpose(Wb[i, :, C:]),
                            bb[i][None, :], gb[i][None, :], btb[i][None, :],
                            feat)

    return jnp.transpose(feat[:N])[None]
